# Initial kernel scaffold; baseline (speedup 1.0000x reference)
import functools

import jax
import jax.numpy as jnp
from jax.experimental import pallas as pl
from jax.experimental.pallas import tpu as pltpu

T = 2048
D = 768
E = 8
F = 1024
SF = 2048
BT = 256


def _silu(x):
    return x * jax.nn.sigmoid(x)


def _moe_block_kernel(x_ref, gw_ref, gb_ref, guw_ref, dw_ref, sguw_ref,
                      sdw_ref, out_ref):
    xb = x_ref[...]

    logits = jax.lax.dot_general(
        xb, gw_ref[...], (((1,), (1,)), ((), ())),
        preferred_element_type=jnp.float32,
        precision=jax.lax.Precision.HIGHEST)
    m = jnp.max(logits, axis=-1, keepdims=True)
    ex = jnp.exp(logits - m)
    probs = ex / jnp.sum(ex, axis=-1, keepdims=True)
    sel = probs + gb_ref[...]

    eids = jax.lax.broadcasted_iota(jnp.int32, (BT, E), 1)
    i0 = jnp.argmax(sel, axis=-1)[:, None]
    sel2 = jnp.where(eids == i0, -jnp.inf, sel)
    i1 = jnp.argmax(sel2, axis=-1)[:, None]
    p0 = jnp.sum(jnp.where(eids == i0, probs, 0.0), axis=-1, keepdims=True)
    p1 = jnp.sum(jnp.where(eids == i1, probs, 0.0), axis=-1, keepdims=True)
    denom = p0 + p1 + 1e-9
    comb = (jnp.where(eids == i0, p0, 0.0)
            + jnp.where(eids == i1, p1, 0.0)) / denom

    xbb = xb.astype(jnp.bfloat16)

    sh = jnp.dot(xbb, sguw_ref[...], preferred_element_type=jnp.float32)
    sa = (_silu(sh[:, :SF]) * sh[:, SF:]).astype(jnp.bfloat16)
    acc = jnp.dot(sa, sdw_ref[...], preferred_element_type=jnp.float32)

    for e in range(E):
        h = jnp.dot(xbb, guw_ref[e], preferred_element_type=jnp.float32)
        a = (_silu(h[:, :F]) * h[:, F:]).astype(jnp.bfloat16)
        eo = jnp.dot(a, dw_ref[e], preferred_element_type=jnp.float32)
        acc = acc + comb[:, e:e + 1] * eo

    out_ref[...] = acc


@jax.jit
def kernel(x, gate_w, gate_bias, gate_up_w, down_w, shared_gate_up_w,
           shared_down_w):
    guw = gate_up_w.astype(jnp.bfloat16)
    dw = down_w.astype(jnp.bfloat16)
    sguw = shared_gate_up_w.astype(jnp.bfloat16)
    sdw = shared_down_w.astype(jnp.bfloat16)
    gb = gate_bias.reshape(1, E)

    grid = (T // BT,)
    full = lambda *s: pl.BlockSpec(s, lambda i: (0,) * len(s))
    out = pl.pallas_call(
        _moe_block_kernel,
        grid=grid,
        in_specs=[
            pl.BlockSpec((BT, D), lambda i: (i, 0)),
            full(E, D),
            full(1, E),
            full(E, D, 2 * F),
            full(E, F, D),
            full(D, 2 * SF),
            full(SF, D),
        ],
        out_specs=pl.BlockSpec((BT, D), lambda i: (i, 0)),
        out_shape=jax.ShapeDtypeStruct((T, D), jnp.float32),
    )(x, gate_w, gb, guw, dw, sguw, sdw)
    return out


# speedup vs baseline: 1.3202x; 1.3202x over previous
import functools

import jax
import jax.numpy as jnp
from jax.experimental import pallas as pl
from jax.experimental.pallas import tpu as pltpu

T = 2048
D = 768
E = 8
F = 1024
SF = 2048
BT = 256


def _silu(x):
    return x * jax.nn.sigmoid(x)


def _moe_block_kernel(x_ref, gw_ref, gb_ref, guw_ref, dw_ref, sguw_ref,
                      sdw_ref, out_ref):
    xb = x_ref[...]
    xbb = xb.astype(jnp.bfloat16)

    logits = jax.lax.dot_general(
        xbb, gw_ref[...].astype(jnp.bfloat16), (((1,), (1,)), ((), ())),
        preferred_element_type=jnp.float32)
    m = jnp.max(logits, axis=-1, keepdims=True)
    ex = jnp.exp(logits - m)
    probs = ex / jnp.sum(ex, axis=-1, keepdims=True)
    sel = probs + gb_ref[...]

    eids = jax.lax.broadcasted_iota(jnp.int32, (BT, E), 1)
    i0 = jnp.argmax(sel, axis=-1)[:, None]
    sel2 = jnp.where(eids == i0, -jnp.inf, sel)
    i1 = jnp.argmax(sel2, axis=-1)[:, None]
    p0 = jnp.sum(jnp.where(eids == i0, probs, 0.0), axis=-1, keepdims=True)
    p1 = jnp.sum(jnp.where(eids == i1, probs, 0.0), axis=-1, keepdims=True)
    denom = p0 + p1 + 1e-9
    comb = (jnp.where(eids == i0, p0, 0.0)
            + jnp.where(eids == i1, p1, 0.0)) / denom

    sh = jnp.dot(xbb, sguw_ref[...], preferred_element_type=jnp.float32)
    sa = (_silu(sh[:, :SF]) * sh[:, SF:]).astype(jnp.bfloat16)
    acc = jnp.dot(sa, sdw_ref[...], preferred_element_type=jnp.float32)

    for e in range(E):
        h = jnp.dot(xbb, guw_ref[e], preferred_element_type=jnp.float32)
        a = (_silu(h[:, :F]) * h[:, F:]).astype(jnp.bfloat16)
        eo = jnp.dot(a, dw_ref[e], preferred_element_type=jnp.float32)
        acc = acc + comb[:, e:e + 1] * eo

    out_ref[...] = acc


@jax.jit
def kernel(x, gate_w, gate_bias, gate_up_w, down_w, shared_gate_up_w,
           shared_down_w):
    guw = gate_up_w.astype(jnp.bfloat16)
    dw = down_w.astype(jnp.bfloat16)
    sguw = shared_gate_up_w.astype(jnp.bfloat16)
    sdw = shared_down_w.astype(jnp.bfloat16)
    gb = gate_bias.reshape(1, E)

    grid = (T // BT,)
    full = lambda *s: pl.BlockSpec(s, lambda i: (0,) * len(s))
    out = pl.pallas_call(
        _moe_block_kernel,
        grid=grid,
        in_specs=[
            pl.BlockSpec((BT, D), lambda i: (i, 0)),
            full(E, D),
            full(1, E),
            full(E, D, 2 * F),
            full(E, F, D),
            full(D, 2 * SF),
            full(SF, D),
        ],
        out_specs=pl.BlockSpec((BT, D), lambda i: (i, 0)),
        out_shape=jax.ShapeDtypeStruct((T, D), jnp.float32),
    )(x, gate_w, gb, guw, dw, sguw, sdw)
    return out
